# Initial kernel scaffold; baseline (speedup 1.0000x reference)
import jax
import jax.numpy as jnp
from jax.experimental import pallas as pl


def kernel(x, edge_index, W1, att_l1, att_r1, b1, W2, att_l2, att_r2, b2):
    raise NotImplementedError("write your pallas kernel here")


# speedup vs baseline: 5.1706x; 5.1706x over previous
import functools

import jax
import jax.numpy as jnp
from jax import lax
from jax.experimental import pallas as pl
from jax.experimental.pallas import tpu as pltpu
from jax.experimental.pallas import tpu_sc as plsc

N_NODES = 10000
N_PAD = 10240
HEADS = 4
D = 128
TROW = 144
NEG = 0.2
EPS = 1e-16

NC = 2
NS = 16
NW = NC * NS
K = 48
N_ACC = 10240


def _mm_att_body(x_ref, w_ref, aw_ref, o_ref, al_ref):
    h = jnp.dot(x_ref[...], w_ref[...], preferred_element_type=jnp.float32)
    o_ref[...] = h
    al_ref[...] = jnp.concatenate(
        [jnp.dot(h, aw_ref[0], preferred_element_type=jnp.float32),
         jnp.dot(h, aw_ref[1], preferred_element_type=jnp.float32),
         jnp.dot(h * h, aw_ref[2], preferred_element_type=jnp.float32)],
        axis=1)


def _mm_att(xp, W, AW, bm=512):
    cin = xp.shape[1]
    cout = W.shape[1]
    grid = (xp.shape[0] // bm,)
    return pl.pallas_call(
        _mm_att_body,
        grid=grid,
        in_specs=[
            pl.BlockSpec((bm, cin), lambda i: (i, 0)),
            pl.BlockSpec((cin, cout), lambda i: (0, 0)),
            pl.BlockSpec((3, cout, 128), lambda i: (0, 0, 0)),
        ],
        out_specs=[
            pl.BlockSpec((bm, cout), lambda i: (i, 0)),
            pl.BlockSpec((bm, 384), lambda i: (i, 0)),
        ],
        out_shape=[
            jax.ShapeDtypeStruct((xp.shape[0], cout), jnp.float32),
            jax.ShapeDtypeStruct((xp.shape[0], 384), jnp.float32),
        ],
    )(xp, W, AW)


def _self_w(al, ar, sq):
    sig = 1.0 / (1.0 + jnp.exp(-sq))
    a = (al + ar) * sig
    a = jnp.where(a >= 0, a, NEG * a)
    return jnp.exp(a)


def _combine1_body(h_ref, alsq_ref, n_ref, d_ref, b_ref, o_ref):
    for h in range(HEADS):
        al = alsq_ref[:, h:h + 1]
        ar = alsq_ref[:, 128 + h:128 + h + 1]
        sq = alsq_ref[:, 256 + h:256 + h + 1]
        ws = _self_w(al, ar, sq)
        hh = h_ref[:, h * D:(h + 1) * D]
        numer = n_ref[h, 0] + n_ref[h, 1] + hh * ws
        denom = d_ref[h, 0, :, 0:1] + d_ref[h, 1, :, 0:1] + ws
        o = numer / (denom + EPS) + b_ref[0, h * D:(h + 1) * D]
        o_ref[:, h * D:(h + 1) * D] = jax.nn.gelu(o)


def _combine1(h1, alsq1, numer, denom, b1, bm=1000):
    n = h1.shape[0]
    grid = (n // bm,)
    return pl.pallas_call(
        _combine1_body,
        grid=grid,
        in_specs=[
            pl.BlockSpec((bm, 4 * D), lambda i: (i, 0)),
            pl.BlockSpec((bm, 384), lambda i: (i, 0)),
            pl.BlockSpec((HEADS, 2, bm, D), lambda i: (0, 0, i, 0)),
            pl.BlockSpec((HEADS, 2, bm, 16), lambda i: (0, 0, i, 0)),
            pl.BlockSpec((1, 4 * D), lambda i: (0, 0)),
        ],
        out_specs=pl.BlockSpec((bm, 4 * D), lambda i: (i, 0)),
        out_shape=jax.ShapeDtypeStruct((n, 4 * D), jnp.float32),
    )(h1, alsq1, numer, denom, b1)


def _combine2_body(h_ref, alsq_ref, n_ref, d_ref, b_ref, o_ref):
    al = alsq_ref[:, 0:1]
    ar = alsq_ref[:, 128:129]
    sq = alsq_ref[:, 256:257]
    ws = _self_w(al, ar, sq)
    numer = n_ref[0] + n_ref[1] + h_ref[...] * ws
    denom = d_ref[0, :, 0:1] + d_ref[1, :, 0:1] + ws
    o_ref[...] = numer / (denom + EPS) + b_ref[0]


def _combine2(h2, alsq2, numer, denom, b2, bm=1000):
    n = h2.shape[0]
    grid = (n // bm,)
    return pl.pallas_call(
        _combine2_body,
        grid=grid,
        in_specs=[
            pl.BlockSpec((bm, D), lambda i: (i, 0)),
            pl.BlockSpec((bm, 384), lambda i: (i, 0)),
            pl.BlockSpec((2, bm, D), lambda i: (0, i, 0)),
            pl.BlockSpec((2, bm, 16), lambda i: (0, i, 0)),
            pl.BlockSpec((1, D), lambda i: (0, 0)),
        ],
        out_specs=pl.BlockSpec((bm, D), lambda i: (i, 0)),
        out_shape=jax.ShapeDtypeStruct((n, D), jnp.float32),
    )(h2, alsq2, numer, denom, b2)


def _edge_pass_body(t_hbm, src_hbm, dst_hbm, z128_hbm, z16_hbm,
                    numer_out, denom_out,
                    sv0, sv1, dv0, dv1, rs0, rs1, rd0, rd1, p_v, w_v,
                    acc, dacc,
                    sem_is0, sem_is1, sem_id0, sem_id1,
                    sem_rs0, sem_rs1, sem_rd0, sem_rd1, n_edges):
    cid = lax.axis_index("c")
    sid = lax.axis_index("s")
    wid = sid * NC + cid
    per_tile = n_edges // NW
    n_blocks = per_tile // K
    rows_per_tile = N_ACC // NS
    base = wid * per_tile

    sv = [sv0, sv1]
    dv = [dv0, dv1]
    rs = [rs0, rs1]
    rd = [rd0, rd1]
    sem_is = [sem_is0, sem_is1]
    sem_id = [sem_id0, sem_id1]
    sem_rs = [sem_rs0, sem_rs1]
    sem_rd = [sem_rd0, sem_rd1]

    ids16 = lax.iota(jnp.int32, 16)
    zero16 = jnp.zeros((16,), jnp.int32)
    fzero16 = jnp.zeros((16,), jnp.float32)

    pltpu.sync_copy(z128_hbm, acc.at[pl.ds(sid * rows_per_tile, rows_per_tile)])
    pltpu.sync_copy(z16_hbm, dacc.at[pl.ds(sid * rows_per_tile, rows_per_tile)])
    def _zw(r, _):
        w_v[r] = fzero16
        return 0
    lax.fori_loop(0, K, _zw, 0)
    plsc.subcore_barrier()

    def idx_start(b, buf):
        off = base + b * K
        pltpu.make_async_copy(src_hbm.at[pl.ds(off, K)], sv[buf], sem_is[buf]).start()
        pltpu.make_async_copy(dst_hbm.at[pl.ds(off, K)], dv[buf], sem_id[buf]).start()

    def idx_wait(b, buf):
        off = base + b * K
        pltpu.make_async_copy(src_hbm.at[pl.ds(off, K)], sv[buf], sem_is[buf]).wait()
        pltpu.make_async_copy(dst_hbm.at[pl.ds(off, K)], dv[buf], sem_id[buf]).wait()

    def rows_start(buf):
        pltpu.make_async_copy(t_hbm.at[sv[buf]], rs[buf], sem_rs[buf]).start()
        pltpu.make_async_copy(t_hbm.at[dv[buf]], rd[buf], sem_rd[buf]).start()

    def rows_wait(buf):
        pltpu.make_async_copy(t_hbm.at[sv[buf]], rs[buf], sem_rs[buf]).wait()
        pltpu.make_async_copy(t_hbm.at[dv[buf]], rd[buf], sem_rd[buf]).wait()

    def compute_scatter(buf):
        rows_s, rows_d = rs[buf], rd[buf]
        for g in range(K // 16):
            lanes = ids16 + (g * 16)

            def dot_body(i, accv):
                for u in range(4):
                    cf = jnp.full((16,), i * 4 + u, jnp.int32)
                    vs = plsc.load_gather(rows_s, [lanes, cf])
                    vd = plsc.load_gather(rows_d, [lanes, cf])
                    accv = accv + vs * vd
                return accv
            logit = lax.fori_loop(0, D // 4, dot_body, fzero16)

            als = plsc.load_gather(rows_s, [lanes, jnp.full((16,), D, jnp.int32)])
            ard = plsc.load_gather(rows_d, [lanes, jnp.full((16,), D + 1, jnp.int32)])
            sig = 1.0 / (1.0 + jnp.exp(-logit))
            a = (als + ard) * sig
            a = jnp.where(a >= 0, a, NEG * a)
            wv = jnp.exp(a)
            plsc.store_scatter(w_v, [lanes, zero16], wv)

            def p_body(i, _):
                for u in range(4):
                    cf = jnp.full((16,), i * 4 + u, jnp.int32)
                    vs = plsc.load_gather(rows_s, [lanes, cf])
                    plsc.store_scatter(p_v, [lanes, cf], vs * wv)
                return 0
            lax.fori_loop(0, D // 4, p_body, 0)

        pltpu.sync_copy(p_v, acc.at[dv[buf]], add=True)
        pltpu.sync_copy(w_v, dacc.at[dv[buf]], add=True)

    idx_start(0, 0)
    idx_start(1, 1)
    idx_wait(0, 0)
    rows_start(0)

    def pair_body(i, _):
        b0 = i * 2
        idx_wait(b0 + 1, 1)
        rows_start(1)
        rows_wait(0)
        compute_scatter(0)

        @pl.when(b0 + 2 < n_blocks)
        def _():
            idx_start(b0 + 2, 0)

        @pl.when(b0 + 2 < n_blocks)
        def _():
            idx_wait(b0 + 2, 0)
            rows_start(0)

        rows_wait(1)
        compute_scatter(1)

        @pl.when(b0 + 3 < n_blocks)
        def _():
            idx_start(b0 + 3, 1)
        return 0

    lax.fori_loop(0, n_blocks // 2, pair_body, 0)
    plsc.subcore_barrier()

    r0 = sid * rows_per_tile
    pltpu.sync_copy(acc.at[pl.ds(r0, rows_per_tile)],
                    numer_out.at[cid, pl.ds(r0, rows_per_tile)])
    pltpu.sync_copy(dacc.at[pl.ds(r0, rows_per_tile)],
                    denom_out.at[cid, pl.ds(r0, rows_per_tile)])


def _edge_pass(table, src, dst, z128, z16):
    n_edges = src.shape[0]
    mesh = plsc.VectorSubcoreMesh(core_axis_name="c", subcore_axis_name="s")
    k = pl.kernel(
        functools.partial(_edge_pass_body, n_edges=n_edges),
        mesh=mesh,
        compiler_params=pltpu.CompilerParams(use_tc_tiling_on_sc=False, needs_layout_passes=False),
        out_type=[
            jax.ShapeDtypeStruct((NC, N_ACC, D), jnp.float32),
            jax.ShapeDtypeStruct((NC, N_ACC, 16), jnp.float32),
        ],
        scratch_types=(
            [pltpu.VMEM((K,), jnp.int32) for _ in range(4)]
            + [pltpu.VMEM((K, TROW), jnp.float32) for _ in range(4)]
            + [pltpu.VMEM((K, D), jnp.float32),
               pltpu.VMEM((K, 16), jnp.float32),
               pltpu.VMEM_SHARED((N_ACC, D), jnp.float32),
               pltpu.VMEM_SHARED((N_ACC, 16), jnp.float32)]
            + [pltpu.SemaphoreType.DMA for _ in range(8)]
        ),
    )
    return k(table, src, dst, z128, z16)


def _attn_weights(att_l, att_r, heads, d):
    eye = jnp.eye(heads, 128, dtype=jnp.float32)
    al = jnp.einsum("hc,hj->hcj", att_l[0], eye).reshape(heads * d, 128)
    ar = jnp.einsum("hc,hj->hcj", att_r[0], eye).reshape(heads * d, 128)
    m = jnp.einsum("hc,hj->hcj", jnp.ones((heads, d), jnp.float32), eye)
    return jnp.stack([al, ar, m.reshape(heads * d, 128)])


def kernel(x, edge_index, W1, att_l1, att_r1, b1, W2, att_l2, att_r2, b2):
    n_edges = edge_index.shape[1]
    blk = NW * K * 2
    e_pad = -(-n_edges // blk) * blk
    src = jnp.pad(edge_index[0].astype(jnp.int32),
                  (0, e_pad - n_edges), constant_values=N_NODES)
    dst = jnp.pad(edge_index[1].astype(jnp.int32),
                  (0, e_pad - n_edges), constant_values=N_NODES)
    xp = jnp.pad(x, ((0, N_PAD - N_NODES), (0, 0)))
    z128 = jnp.zeros((N_ACC // NS, D), jnp.float32)
    z16 = jnp.zeros((N_ACC // NS, 16), jnp.float32)
    zpad = jnp.zeros((N_NODES, TROW - D - 2), jnp.float32)

    h1p, alsq1p = _mm_att(xp, W1, _attn_weights(att_l1, att_r1, HEADS, D))
    h1 = h1p[:N_NODES]
    alsq1 = alsq1p[:N_NODES]
    numers, denoms = [], []
    for h in range(HEADS):
        table = jnp.pad(jnp.concatenate(
            [h1[:, h * D:(h + 1) * D], alsq1[:, h:h + 1],
             alsq1[:, 128 + h:128 + h + 1], zpad], axis=1),
            ((0, N_ACC - N_NODES), (0, 0)))
        n_p, d_p = _edge_pass(table, src, dst, z128, z16)
        numers.append(n_p)
        denoms.append(d_p)
    x2 = _combine1(h1, alsq1, jnp.stack(numers), jnp.stack(denoms),
                   b1.reshape(1, -1))

    x2p = jnp.pad(x2, ((0, N_PAD - N_NODES), (0, 0)))
    h2p, alsq2p = _mm_att(x2p, W2, _attn_weights(att_l2, att_r2, 1, D))
    h2 = h2p[:N_NODES]
    alsq2 = alsq2p[:N_NODES]
    table2 = jnp.pad(jnp.concatenate(
        [h2, alsq2[:, 0:1], alsq2[:, 128:129], zpad], axis=1),
        ((0, N_ACC - N_NODES), (0, 0)))
    n_p, d_p = _edge_pass(table2, src, dst, z128, z16)
    return _combine2(h2, alsq2, n_p, d_p, b2.reshape(1, -1))


# speedup vs baseline: 6.9540x; 1.3449x over previous
import functools

import jax
import jax.numpy as jnp
from jax import lax
from jax.experimental import pallas as pl
from jax.experimental.pallas import tpu as pltpu
from jax.experimental.pallas import tpu_sc as plsc

N_NODES = 10000
N_PAD = 10240
HEADS = 4
D = 128
TROW = 144
NEG = 0.2
EPS = 1e-16

NC = 2
NS = 16
NW = NC * NS
K = 48
N_ACC = 10240


def _mm_att_body(x_ref, w_ref, aw_ref, o_ref, al_ref):
    h = jnp.dot(x_ref[...], w_ref[...], preferred_element_type=jnp.float32)
    o_ref[...] = h
    al_ref[...] = jnp.concatenate(
        [jnp.dot(h, aw_ref[0], preferred_element_type=jnp.float32),
         jnp.dot(h, aw_ref[1], preferred_element_type=jnp.float32),
         jnp.dot(h * h, aw_ref[2], preferred_element_type=jnp.float32)],
        axis=1)


def _mm_att(xp, W, AW, bm=512):
    cin = xp.shape[1]
    cout = W.shape[1]
    grid = (xp.shape[0] // bm,)
    return pl.pallas_call(
        _mm_att_body,
        grid=grid,
        in_specs=[
            pl.BlockSpec((bm, cin), lambda i: (i, 0)),
            pl.BlockSpec((cin, cout), lambda i: (0, 0)),
            pl.BlockSpec((3, cout, 128), lambda i: (0, 0, 0)),
        ],
        out_specs=[
            pl.BlockSpec((bm, cout), lambda i: (i, 0)),
            pl.BlockSpec((bm, 384), lambda i: (i, 0)),
        ],
        out_shape=[
            jax.ShapeDtypeStruct((xp.shape[0], cout), jnp.float32),
            jax.ShapeDtypeStruct((xp.shape[0], 384), jnp.float32),
        ],
    )(xp, W, AW)


def _self_w(al, ar, sq):
    sig = 1.0 / (1.0 + jnp.exp(-sq))
    a = (al + ar) * sig
    a = jnp.where(a >= 0, a, NEG * a)
    return jnp.exp(a)


def _combine1_body(h_ref, alsq_ref, n_ref, b_ref, o_ref):
    for h in range(HEADS):
        al = alsq_ref[:, h:h + 1]
        ar = alsq_ref[:, 128 + h:128 + h + 1]
        sq = alsq_ref[:, 256 + h:256 + h + 1]
        ws = _self_w(al, ar, sq)
        hh = h_ref[:, h * D:(h + 1) * D]
        numer = n_ref[h, 0, :, :D] + n_ref[h, 1, :, :D] + hh * ws
        denom = n_ref[h, 0, :, D:D + 1] + n_ref[h, 1, :, D:D + 1] + ws
        o = numer / (denom + EPS) + b_ref[0, h * D:(h + 1) * D]
        o_ref[:, h * D:(h + 1) * D] = jax.nn.gelu(o)


def _combine1(h1, alsq1, numer, b1, bm=1000):
    n = h1.shape[0]
    grid = (n // bm,)
    return pl.pallas_call(
        _combine1_body,
        grid=grid,
        in_specs=[
            pl.BlockSpec((bm, 4 * D), lambda i: (i, 0)),
            pl.BlockSpec((bm, 384), lambda i: (i, 0)),
            pl.BlockSpec((HEADS, 2, bm, TROW), lambda i: (0, 0, i, 0)),
            pl.BlockSpec((1, 4 * D), lambda i: (0, 0)),
        ],
        out_specs=pl.BlockSpec((bm, 4 * D), lambda i: (i, 0)),
        out_shape=jax.ShapeDtypeStruct((n, 4 * D), jnp.float32),
    )(h1, alsq1, numer, b1)


def _combine2_body(h_ref, alsq_ref, n_ref, b_ref, o_ref):
    al = alsq_ref[:, 0:1]
    ar = alsq_ref[:, 128:129]
    sq = alsq_ref[:, 256:257]
    ws = _self_w(al, ar, sq)
    numer = n_ref[0, :, :D] + n_ref[1, :, :D] + h_ref[...] * ws
    denom = n_ref[0, :, D:D + 1] + n_ref[1, :, D:D + 1] + ws
    o_ref[...] = numer / (denom + EPS) + b_ref[0]


def _combine2(h2, alsq2, numer, b2, bm=1000):
    n = h2.shape[0]
    grid = (n // bm,)
    return pl.pallas_call(
        _combine2_body,
        grid=grid,
        in_specs=[
            pl.BlockSpec((bm, D), lambda i: (i, 0)),
            pl.BlockSpec((bm, 384), lambda i: (i, 0)),
            pl.BlockSpec((2, bm, TROW), lambda i: (0, i, 0)),
            pl.BlockSpec((1, D), lambda i: (0, 0)),
        ],
        out_specs=pl.BlockSpec((bm, D), lambda i: (i, 0)),
        out_shape=jax.ShapeDtypeStruct((n, D), jnp.float32),
    )(h2, alsq2, numer, b2)


def _edge_pass_body(t_hbm, src_hbm, dst_hbm, z128_hbm,
                    numer_out,
                    sv0, sv1, dv0, dv1, rs0, rs1, rd0, rd1,
                    acc,
                    sem_is0, sem_is1, sem_id0, sem_id1,
                    sem_rs0, sem_rs1, sem_rd0, sem_rd1, n_edges):
    cid = lax.axis_index("c")
    sid = lax.axis_index("s")
    wid = sid * NC + cid
    per_tile = n_edges // NW
    n_blocks = per_tile // K
    rows_per_tile = N_ACC // NS
    base = wid * per_tile

    sv = [sv0, sv1]
    dv = [dv0, dv1]
    rs = [rs0, rs1]
    rd = [rd0, rd1]
    sem_is = [sem_is0, sem_is1]
    sem_id = [sem_id0, sem_id1]
    sem_rs = [sem_rs0, sem_rs1]
    sem_rd = [sem_rd0, sem_rd1]

    ids16 = lax.iota(jnp.int32, 16)
    zero16 = jnp.zeros((16,), jnp.int32)
    fzero16 = jnp.zeros((16,), jnp.float32)

    pltpu.sync_copy(z128_hbm, acc.at[pl.ds(sid * rows_per_tile, rows_per_tile)])
    plsc.subcore_barrier()

    def idx_start(b, buf):
        off = base + b * K
        pltpu.make_async_copy(src_hbm.at[pl.ds(off, K)], sv[buf], sem_is[buf]).start()
        pltpu.make_async_copy(dst_hbm.at[pl.ds(off, K)], dv[buf], sem_id[buf]).start()

    def idx_wait(b, buf):
        off = base + b * K
        pltpu.make_async_copy(src_hbm.at[pl.ds(off, K)], sv[buf], sem_is[buf]).wait()
        pltpu.make_async_copy(dst_hbm.at[pl.ds(off, K)], dv[buf], sem_id[buf]).wait()

    def rows_start(buf):
        pltpu.make_async_copy(t_hbm.at[sv[buf]], rs[buf], sem_rs[buf]).start()
        pltpu.make_async_copy(t_hbm.at[dv[buf]], rd[buf], sem_rd[buf]).start()

    def rows_wait(buf):
        pltpu.make_async_copy(t_hbm.at[sv[buf]], rs[buf], sem_rs[buf]).wait()
        pltpu.make_async_copy(t_hbm.at[dv[buf]], rd[buf], sem_rd[buf]).wait()

    def compute_scatter(buf):
        rows_s, rows_d = rs[buf], rd[buf]
        for g in range(K // 16):
            lanes = ids16 + (g * 16)

            def dot_body(i, accv):
                for u in range(4):
                    cf = jnp.full((16,), i * 4 + u, jnp.int32)
                    vs = plsc.load_gather(rows_s, [lanes, cf])
                    vd = plsc.load_gather(rows_d, [lanes, cf])
                    accv = accv + vs * vd
                return accv
            logit = lax.fori_loop(0, D // 4, dot_body, fzero16)

            als = plsc.load_gather(rows_s, [lanes, jnp.full((16,), D, jnp.int32)])
            ard = plsc.load_gather(rows_d, [lanes, jnp.full((16,), D + 1, jnp.int32)])
            sig = 1.0 / (1.0 + jnp.exp(-logit))
            a = (als + ard) * sig
            a = jnp.where(a >= 0, a, NEG * a)
            wv = jnp.exp(a)

            def p_body(i, _):
                for u in range(4):
                    cf = jnp.full((16,), i * 4 + u, jnp.int32)
                    vs = plsc.load_gather(rows_s, [lanes, cf])
                    plsc.store_scatter(rows_s, [lanes, cf], vs * wv)
                return 0
            lax.fori_loop(0, D // 4, p_body, 0)
            plsc.store_scatter(rows_s, [lanes, jnp.full((16,), D, jnp.int32)], wv)

        pltpu.sync_copy(rs[buf], acc.at[dv[buf]], add=True)

    idx_start(0, 0)
    idx_start(1, 1)
    idx_wait(0, 0)
    rows_start(0)

    def pair_body(i, _):
        b0 = i * 2
        idx_wait(b0 + 1, 1)
        rows_start(1)
        rows_wait(0)
        compute_scatter(0)

        @pl.when(b0 + 2 < n_blocks)
        def _():
            idx_start(b0 + 2, 0)

        @pl.when(b0 + 2 < n_blocks)
        def _():
            idx_wait(b0 + 2, 0)
            rows_start(0)

        rows_wait(1)
        compute_scatter(1)

        @pl.when(b0 + 3 < n_blocks)
        def _():
            idx_start(b0 + 3, 1)
        return 0

    lax.fori_loop(0, n_blocks // 2, pair_body, 0)
    plsc.subcore_barrier()

    r0 = sid * rows_per_tile
    pltpu.sync_copy(acc.at[pl.ds(r0, rows_per_tile)],
                    numer_out.at[cid, pl.ds(r0, rows_per_tile)])


def _edge_pass(table, src, dst, z128):
    n_edges = src.shape[0]
    mesh = plsc.VectorSubcoreMesh(core_axis_name="c", subcore_axis_name="s")
    k = pl.kernel(
        functools.partial(_edge_pass_body, n_edges=n_edges),
        mesh=mesh,
        compiler_params=pltpu.CompilerParams(use_tc_tiling_on_sc=False, needs_layout_passes=False),
        out_type=jax.ShapeDtypeStruct((NC, N_ACC, TROW), jnp.float32),
        scratch_types=(
            [pltpu.VMEM((K,), jnp.int32) for _ in range(4)]
            + [pltpu.VMEM((K, TROW), jnp.float32) for _ in range(4)]
            + [pltpu.VMEM_SHARED((N_ACC, TROW), jnp.float32)]
            + [pltpu.SemaphoreType.DMA for _ in range(8)]
        ),
    )
    return k(table, src, dst, z128)


def _attn_weights(att_l, att_r, heads, d):
    eye = jnp.eye(heads, 128, dtype=jnp.float32)
    al = jnp.einsum("hc,hj->hcj", att_l[0], eye).reshape(heads * d, 128)
    ar = jnp.einsum("hc,hj->hcj", att_r[0], eye).reshape(heads * d, 128)
    m = jnp.einsum("hc,hj->hcj", jnp.ones((heads, d), jnp.float32), eye)
    return jnp.stack([al, ar, m.reshape(heads * d, 128)])


def kernel(x, edge_index, W1, att_l1, att_r1, b1, W2, att_l2, att_r2, b2):
    n_edges = edge_index.shape[1]
    blk = NW * K * 2
    e_pad = -(-n_edges // blk) * blk
    src = jnp.pad(edge_index[0].astype(jnp.int32),
                  (0, e_pad - n_edges), constant_values=N_NODES)
    dst = jnp.pad(edge_index[1].astype(jnp.int32),
                  (0, e_pad - n_edges), constant_values=N_NODES)
    xp = jnp.pad(x, ((0, N_PAD - N_NODES), (0, 0)))
    z128 = jnp.zeros((N_ACC // NS, TROW), jnp.float32)
    zpad = jnp.zeros((N_NODES, TROW - D - 2), jnp.float32)

    h1p, alsq1p = _mm_att(xp, W1, _attn_weights(att_l1, att_r1, HEADS, D))
    h1 = h1p[:N_NODES]
    alsq1 = alsq1p[:N_NODES]
    numers = []
    for h in range(HEADS):
        table = jnp.pad(jnp.concatenate(
            [h1[:, h * D:(h + 1) * D], alsq1[:, h:h + 1],
             alsq1[:, 128 + h:128 + h + 1], zpad], axis=1),
            ((0, N_ACC - N_NODES), (0, 0)))
        numers.append(_edge_pass(table, src, dst, z128))
    x2 = _combine1(h1, alsq1, jnp.stack(numers), b1.reshape(1, -1))

    x2p = jnp.pad(x2, ((0, N_PAD - N_NODES), (0, 0)))
    h2p, alsq2p = _mm_att(x2p, W2, _attn_weights(att_l2, att_r2, 1, D))
    h2 = h2p[:N_NODES]
    alsq2 = alsq2p[:N_NODES]
    table2 = jnp.pad(jnp.concatenate(
        [h2, alsq2[:, 0:1], alsq2[:, 128:129], zpad], axis=1),
        ((0, N_ACC - N_NODES), (0, 0)))
    n_p = _edge_pass(table2, src, dst, z128)
    return _combine2(h2, alsq2, n_p, b2.reshape(1, -1))
